# Initial kernel scaffold; baseline (speedup 1.0000x reference)
import jax
import jax.numpy as jnp
from jax.experimental import pallas as pl


def kernel(stage, x, act, W1, b1, Wv, bv, Wa, ba, log_std):
    raise NotImplementedError("write your pallas kernel here")


# speedup vs baseline: 1.1780x; 1.1780x over previous
import functools

import jax
import jax.numpy as jnp
import numpy as np
from jax.experimental import pallas as pl

N = 8192
D = 1024
H = 2048
A = 64

ROW_BLK = 256
_LOG2PI = float(np.log(2.0 * np.pi))


def _fused_kernel(stage_ref, x_ref, act_ref, w1_ref, b1_ref, w2_ref, b2_ref,
                  logstd_ref, val_ref, logp_ref, ent_ref):
    i = pl.program_id(0)
    nsteps = pl.num_programs(0)

    x = x_ref[...]
    h = jnp.tanh(jnp.dot(x, w1_ref[...], preferred_element_type=jnp.float32)
                 + b1_ref[...])
    out2 = jnp.dot(h, w2_ref[...], preferred_element_type=jnp.float32) + b2_ref[...]
    val = out2[:, 0:1]
    mu = out2[:, 1:1 + A]

    log_std = logstd_ref[...]
    inv_std = jnp.exp(-log_std)
    diff = (act_ref[...] - mu) * inv_std
    sum_log_std = jnp.sum(log_std)
    logp = (-0.5 * jnp.sum(diff * diff, axis=-1, keepdims=True)
            - sum_log_std - 0.5 * A * _LOG2PI)

    m = (stage_ref[...] > 0).astype(jnp.float32)
    val_ref[...] = val * m
    logp_ref[...] = logp * m

    cnt = jnp.sum(m).reshape(1, 1)

    @pl.when(i == 0)
    def _init():
        ent_ref[...] = jnp.zeros((1, 1), jnp.float32)

    ent_ref[...] += cnt

    @pl.when(i == nsteps - 1)
    def _fini():
        ent_scalar = sum_log_std + 0.5 * A * (_LOG2PI + 1.0)
        ent_ref[...] = ent_ref[...] * (ent_scalar / N)


@functools.partial(jax.jit, static_argnames=())
def kernel(stage, x, act, W1, b1, Wv, bv, Wa, ba, log_std):
    stage2 = stage.reshape(N, 1).astype(jnp.int32)
    W2 = jnp.concatenate([Wv, Wa], axis=1)
    b2 = jnp.concatenate([bv, ba]).reshape(1, 1 + A)
    b1r = b1.reshape(1, H)
    lsr = log_std.reshape(1, A)

    grid = (N // ROW_BLK,)
    val, logp, ent = pl.pallas_call(
        _fused_kernel,
        grid=grid,
        in_specs=[
            pl.BlockSpec((ROW_BLK, 1), lambda i: (i, 0)),
            pl.BlockSpec((ROW_BLK, D), lambda i: (i, 0)),
            pl.BlockSpec((ROW_BLK, A), lambda i: (i, 0)),
            pl.BlockSpec((D, H), lambda i: (0, 0)),
            pl.BlockSpec((1, H), lambda i: (0, 0)),
            pl.BlockSpec((H, 1 + A), lambda i: (0, 0)),
            pl.BlockSpec((1, 1 + A), lambda i: (0, 0)),
            pl.BlockSpec((1, A), lambda i: (0, 0)),
        ],
        out_specs=[
            pl.BlockSpec((ROW_BLK, 1), lambda i: (i, 0)),
            pl.BlockSpec((ROW_BLK, 1), lambda i: (i, 0)),
            pl.BlockSpec((1, 1), lambda i: (0, 0)),
        ],
        out_shape=[
            jax.ShapeDtypeStruct((N, 1), jnp.float32),
            jax.ShapeDtypeStruct((N, 1), jnp.float32),
            jax.ShapeDtypeStruct((1, 1), jnp.float32),
        ],
    )(stage2, x, act, W1, b1r, W2, b2, lsr)
    return (val, logp, ent[0, 0])
